# baseline (speedup 1.0000x reference)
import functools

import jax
import jax.numpy as jnp
from jax import lax
from jax.experimental import pallas as pl
from jax.experimental.pallas import tpu as pltpu
from jax.experimental.pallas import tpu_sc as plsc

N = 10000
E = 320000
DF = 128
DE = 16
HE = 128
HN = 128

K = 2
E2 = E // K
NC = 2
NS = 16
NW = NC * NS
EPW = E2 // NW
CB = 40
NCH = EPW // CB
NP = 10240
RPT = NP // NS

EB = 3200
NB = 2000

_LOG2E = 1.4426950408889634
_LN2 = 0.6931471805599453


def _softplus(x):
    p = jnp.exp2(jnp.abs(x) * -_LOG2E)
    return jnp.maximum(x, 0.0) + jnp.log2(1.0 + p) * _LN2


def _precomp_body(x_ref, wa_ref, wb_ref, p1_ref, p2_ref):
    xb = x_ref[...]
    p1_ref[...] = jnp.dot(xb, wa_ref[...], preferred_element_type=jnp.float32)
    p2_ref[...] = jnp.dot(xb, wb_ref[...], preferred_element_type=jnp.float32)


def _edge_mlp_inner(g1_ref, g2_ref, ea_ref, w1c_ref, b1_ref, w2_ref,
                    b2_ref, w3_ref, b3_ref, he_ref):
    h = (g1_ref[...] + g2_ref[...]
         + jnp.dot(ea_ref[...], w1c_ref[...], preferred_element_type=jnp.float32)
         + b1_ref[...])
    h = _softplus(h)
    h = _softplus(jnp.dot(h, w2_ref[...], preferred_element_type=jnp.float32)
                  + b2_ref[...])
    he_ref[...] = (jnp.dot(h, w3_ref[...], preferred_element_type=jnp.float32)
                   + b3_ref[...])


def _edge_mlp_body0(g1_ref, g2_ref, ea_ref, w1c_ref, b1_ref, w2_ref,
                    b2_ref, w3_ref, b3_ref, he_ref):
    _edge_mlp_inner(g1_ref, g2_ref, ea_ref, w1c_ref, b1_ref, w2_ref,
                    b2_ref, w3_ref, b3_ref, he_ref)


def _edge_mlp_body1(he_in_ref, g1_ref, g2_ref, ea_ref, w1c_ref, b1_ref,
                    w2_ref, b2_ref, w3_ref, b3_ref, he_ref):
    del he_in_ref
    _edge_mlp_inner(g1_ref, g2_ref, ea_ref, w1c_ref, b1_ref, w2_ref,
                    b2_ref, w3_ref, b3_ref, he_ref)


def _node_mlp_body(x_ref, a0_ref, a1_ref, wna_ref, wnb_ref, b1_ref, w2_ref,
                   b2_ref, w3_ref, b3_ref, hn_ref):
    a = (a0_ref[0] + a0_ref[1]) + (a1_ref[0] + a1_ref[1])
    g = (jnp.dot(x_ref[...], wna_ref[...], preferred_element_type=jnp.float32)
         + jnp.dot(a, wnb_ref[...], preferred_element_type=jnp.float32)
         + b1_ref[...])
    g = _softplus(g)
    g = _softplus(jnp.dot(g, w2_ref[...], preferred_element_type=jnp.float32)
                  + b2_ref[...])
    hn_ref[...] = (jnp.dot(g, w3_ref[...], preferred_element_type=jnp.float32)
                   + b3_ref[...])


def _gather_body(p1, p2, srcr, dstr, g1, g2,
                 idx_s, idx_d, b1a, b1b, b2a, b2b,
                 g1a, g1b, g2a, g2b, w1a, w1b, w2a, w2b):
    cid = lax.axis_index("c")
    sid = lax.axis_index("s")
    wid = sid * NC + cid
    base = wid * EPW
    pltpu.sync_copy(srcr.at[wid], idx_s)
    pltpu.sync_copy(dstr.at[wid], idx_d)
    bufs1 = (b1a, b1b)
    bufs2 = (b2a, b2b)
    gsem1 = (g1a, g1b)
    gsem2 = (g2a, g2b)
    wsem1 = (w1a, w1b)
    wsem2 = (w2a, w2b)
    for b in range(2):
        pltpu.async_copy(p1.at[idx_s.at[b]], bufs1[b], gsem1[b])
        pltpu.async_copy(p2.at[idx_d.at[b]], bufs2[b], gsem2[b])

    def step(t, carry):
        j0 = 2 * t
        for b in range(2):
            j = j0 + b
            pltpu.make_async_copy(p1.at[idx_s.at[j]], bufs1[b], gsem1[b]).wait()
            pltpu.async_copy(bufs1[b], g1.at[pl.ds(base + j * CB, CB)], wsem1[b])
            pltpu.make_async_copy(p2.at[idx_d.at[j]], bufs2[b], gsem2[b]).wait()
            pltpu.async_copy(bufs2[b], g2.at[pl.ds(base + j * CB, CB)], wsem2[b])

            @pl.when(j + 2 < NCH)
            def _():
                pltpu.make_async_copy(bufs1[b], g1.at[pl.ds(base + j * CB, CB)],
                                      wsem1[b]).wait()
                pltpu.async_copy(p1.at[idx_s.at[j + 2]], bufs1[b], gsem1[b])
                pltpu.make_async_copy(bufs2[b], g2.at[pl.ds(base + j * CB, CB)],
                                      wsem2[b]).wait()
                pltpu.async_copy(p2.at[idx_d.at[j + 2]], bufs2[b], gsem2[b])
        return carry

    lax.fori_loop(0, (NCH - 1) // 2, step, 0)
    j = NCH - 2
    pltpu.make_async_copy(b1b, g1.at[pl.ds(base + j * CB, CB)], wsem1[1]).wait()
    pltpu.make_async_copy(b2b, g2.at[pl.ds(base + j * CB, CB)], wsem2[1]).wait()
    j = NCH - 1
    pltpu.make_async_copy(p1.at[idx_s.at[j]], b1a, gsem1[0]).wait()
    pltpu.sync_copy(b1a, g1.at[pl.ds(base + j * CB, CB)])
    pltpu.make_async_copy(p2.at[idx_d.at[j]], b2a, gsem2[0]).wait()
    pltpu.sync_copy(b2a, g2.at[pl.ds(base + j * CB, CB)])


def _make_scatter_body(kbase):
    def _scatter_body(he, dstr, zz, aggp, idx_d, ba, bb, la, lb, xa, xb, acc):
        cid = lax.axis_index("c")
        sid = lax.axis_index("s")
        wid = sid * NC + cid
        base = kbase + wid * EPW
        pltpu.sync_copy(dstr.at[wid], idx_d)
        pltpu.sync_copy(zz.at[pl.ds(sid * RPT, RPT)],
                        acc.at[pl.ds(sid * RPT, RPT)])
        plsc.subcore_barrier()
        bufs = (ba, bb)
        lsem = (la, lb)
        xsem = (xa, xb)
        for b in range(2):
            pltpu.async_copy(he.at[pl.ds(base + b * CB, CB)], bufs[b], lsem[b])

        def step(t, carry):
            j0 = 2 * t
            for b in range(2):
                j = j0 + b
                pltpu.make_async_copy(he.at[pl.ds(base + j * CB, CB)],
                                      bufs[b], lsem[b]).wait()
                pltpu.async_copy(bufs[b], acc.at[idx_d.at[j]], xsem[b],
                                 add=True)

                @pl.when(j + 2 < NCH)
                def _():
                    pltpu.make_async_copy(bufs[b], acc.at[idx_d.at[j]],
                                          xsem[b]).wait()
                    pltpu.async_copy(he.at[pl.ds(base + (j + 2) * CB, CB)],
                                     bufs[b], lsem[b])
            return carry

        lax.fori_loop(0, (NCH - 1) // 2, step, 0)
        pltpu.make_async_copy(bb, acc.at[idx_d.at[NCH - 2]], xsem[1]).wait()
        j = NCH - 1
        pltpu.make_async_copy(he.at[pl.ds(base + j * CB, CB)], ba,
                              lsem[0]).wait()
        pltpu.sync_copy(ba, acc.at[idx_d.at[j]], add=True)
        plsc.subcore_barrier()
        pltpu.sync_copy(acc.at[pl.ds(sid * RPT, RPT)],
                        aggp.at[cid, pl.ds(sid * RPT, RPT)])

    return _scatter_body


_SC_MESH = plsc.VectorSubcoreMesh(core_axis_name="c", subcore_axis_name="s")

_gather_call = functools.partial(
    pl.kernel,
    mesh=_SC_MESH,
    out_type=(jax.ShapeDtypeStruct((E2, HE), jnp.float32),
              jax.ShapeDtypeStruct((E2, HE), jnp.float32)),
    scratch_types=[
        pltpu.VMEM((NCH, CB), jnp.int32),
        pltpu.VMEM((NCH, CB), jnp.int32),
        pltpu.VMEM((CB, HE), jnp.float32),
        pltpu.VMEM((CB, HE), jnp.float32),
        pltpu.VMEM((CB, HE), jnp.float32),
        pltpu.VMEM((CB, HE), jnp.float32),
        pltpu.SemaphoreType.DMA,
        pltpu.SemaphoreType.DMA,
        pltpu.SemaphoreType.DMA,
        pltpu.SemaphoreType.DMA,
        pltpu.SemaphoreType.DMA,
        pltpu.SemaphoreType.DMA,
        pltpu.SemaphoreType.DMA,
        pltpu.SemaphoreType.DMA,
    ],
)(_gather_body)

_scatter_calls = tuple(
    functools.partial(
        pl.kernel,
        mesh=_SC_MESH,
        out_type=jax.ShapeDtypeStruct((NC, NP, HE), jnp.float32),
        scratch_types=[
            pltpu.VMEM((NCH, CB), jnp.int32),
            pltpu.VMEM((CB, HE), jnp.float32),
            pltpu.VMEM((CB, HE), jnp.float32),
            pltpu.SemaphoreType.DMA,
            pltpu.SemaphoreType.DMA,
            pltpu.SemaphoreType.DMA,
            pltpu.SemaphoreType.DMA,
            pltpu.VMEM_SHARED((NP, HE), jnp.float32),
        ],
    )(_make_scatter_body(k * E2))
    for k in range(K)
)


def kernel(edge_index, x, edge_attr, We1, be1, We2, be2, We3, be3,
           Wn1, bn1, Wn2, bn2, Wn3, bn3):
    We1a = We1[:DF]
    We1b = We1[DF:2 * DF]
    We1c = We1[2 * DF:]
    Wn1a = Wn1[:DF]
    Wn1b = Wn1[DF:]
    srcr = edge_index[0].reshape(K, NW, NCH, CB)
    dstr = edge_index[1].reshape(K, NW, NCH, CB)
    zeros = jnp.zeros((NP, HE), jnp.float32)
    be1r = be1.reshape(1, HE)
    be2r = be2.reshape(1, HE)
    be3r = be3.reshape(1, HE)
    bn1r = bn1.reshape(1, HN)
    bn2r = bn2.reshape(1, HN)
    bn3r = bn3.reshape(1, HN)

    p1, p2 = pl.pallas_call(
        _precomp_body,
        grid=(N // NB,),
        in_specs=[
            pl.BlockSpec((NB, DF), lambda i: (i, 0)),
            pl.BlockSpec((DF, HE), lambda i: (0, 0)),
            pl.BlockSpec((DF, HE), lambda i: (0, 0)),
        ],
        out_specs=[
            pl.BlockSpec((NB, HE), lambda i: (i, 0)),
            pl.BlockSpec((NB, HE), lambda i: (i, 0)),
        ],
        out_shape=[
            jax.ShapeDtypeStruct((N, HE), jnp.float32),
            jax.ShapeDtypeStruct((N, HE), jnp.float32),
        ],
    )(x, We1a, We1b)

    gs = [_gather_call(p1, p2, srcr[k], dstr[k]) for k in range(K)]

    NBLK = E2 // EB
    he = None
    for k in range(K):
        g1k, g2k = gs[k]
        eak = lax.slice_in_dim(edge_attr, k * E2, (k + 1) * E2)
        data_specs = [
            pl.BlockSpec((EB, HE), lambda i: (i, 0)),
            pl.BlockSpec((EB, HE), lambda i: (i, 0)),
            pl.BlockSpec((EB, DE), lambda i: (i, 0)),
            pl.BlockSpec((DE, HE), lambda i: (0, 0)),
            pl.BlockSpec((1, HE), lambda i: (0, 0)),
            pl.BlockSpec((HE, HE), lambda i: (0, 0)),
            pl.BlockSpec((1, HE), lambda i: (0, 0)),
            pl.BlockSpec((HE, HE), lambda i: (0, 0)),
            pl.BlockSpec((1, HE), lambda i: (0, 0)),
        ]
        args = (g1k, g2k, eak, We1c, be1r, We2, be2r, We3, be3r)
        if k == 0:
            he = pl.pallas_call(
                _edge_mlp_body0,
                grid=(NBLK,),
                in_specs=data_specs,
                out_specs=pl.BlockSpec((EB, HE), lambda i: (i, 0)),
                out_shape=jax.ShapeDtypeStruct((E, HE), jnp.float32),
            )(*args)
        else:
            he = pl.pallas_call(
                _edge_mlp_body1,
                grid=(NBLK,),
                in_specs=[pl.BlockSpec(memory_space=pl.ANY)] + data_specs,
                out_specs=pl.BlockSpec((EB, HE),
                                       lambda i, k=k: (k * NBLK + i, 0)),
                out_shape=jax.ShapeDtypeStruct((E, HE), jnp.float32),
                input_output_aliases={0: 0},
            )(he, *args)

    aggs = [_scatter_calls[k](he, dstr[k], zeros) for k in range(K)]

    h_n = pl.pallas_call(
        _node_mlp_body,
        grid=(N // NB,),
        in_specs=[
            pl.BlockSpec((NB, DF), lambda i: (i, 0)),
            pl.BlockSpec((NC, NB, HE), lambda i: (0, i, 0)),
            pl.BlockSpec((NC, NB, HE), lambda i: (0, i, 0)),
            pl.BlockSpec((DF, HN), lambda i: (0, 0)),
            pl.BlockSpec((HE, HN), lambda i: (0, 0)),
            pl.BlockSpec((1, HN), lambda i: (0, 0)),
            pl.BlockSpec((HN, HN), lambda i: (0, 0)),
            pl.BlockSpec((1, HN), lambda i: (0, 0)),
            pl.BlockSpec((HN, HN), lambda i: (0, 0)),
            pl.BlockSpec((1, HN), lambda i: (0, 0)),
        ],
        out_specs=pl.BlockSpec((NB, HN), lambda i: (i, 0)),
        out_shape=jax.ShapeDtypeStruct((N, HN), jnp.float32),
    )(x, aggs[0], aggs[1], Wn1a, Wn1b, bn1r, Wn2, bn2r, Wn3, bn3r)

    return (he, h_n)
